# baseline (speedup 1.0000x reference)
import jax
import jax.numpy as jnp
from jax import lax
from jax.experimental import pallas as pl
from jax.experimental.pallas import tpu as pltpu

N0, N1, K = 4096, 20, 1000
B = 64
NBUF = 4
NSTEPS = N0 // B


def _onehot_body(values_ref, idx_ref, out_hbm, *bufs_and_sems):
    bufs = bufs_and_sems[:NBUF]
    sems = bufs_and_sems[NBUF]
    i = pl.program_id(0)

    def _copy(step, s):
        return pltpu.make_async_copy(
            bufs[s],
            out_hbm.at[pl.ds(step * B, B)],
            sems.at[s],
        )

    off = values_ref[0]
    on = values_ref[1]
    idx = idx_ref[...]
    kk = lax.broadcasted_iota(jnp.int32, (B, N1, K), 2)
    block = jnp.where(kk == idx, on, off)

    for s in range(NBUF):
        @pl.when(lax.rem(i, NBUF) == s)
        def _():
            @pl.when(i >= NBUF)
            def _():
                _copy(i - NBUF, s).wait()

            bufs[s][...] = block
            _copy(i, s).start()

    @pl.when(i == NSTEPS - 1)
    def _():
        for j in range(NBUF - 1, -1, -1):
            step = NSTEPS - 1 - j
            for s in range(NBUF):
                @pl.when(lax.rem(step, NBUF) == s)
                def _():
                    _copy(step, s).wait()


def kernel(indices, values):
    return pl.pallas_call(
        _onehot_body,
        grid=(NSTEPS,),
        in_specs=[
            pl.BlockSpec(memory_space=pltpu.SMEM),
            pl.BlockSpec((B, N1, 1), lambda i: (i, 0, 0)),
        ],
        out_specs=pl.BlockSpec(memory_space=pl.ANY),
        out_shape=jax.ShapeDtypeStruct((N0, N1, K), jnp.float32),
        scratch_shapes=[pltpu.VMEM((B, N1, K), jnp.float32) for _ in range(NBUF)]
        + [pltpu.SemaphoreType.DMA((NBUF,))],
    )(values, indices.reshape(N0, N1, 1))


# speedup vs baseline: 1.0203x; 1.0203x over previous
import dataclasses

import jax
import jax.numpy as jnp
from jax import lax
from jax.experimental import pallas as pl
from jax.experimental.pallas import tpu as pltpu
from jax.experimental.pallas import tpu_sc as plsc

N0, N1, K = 4096, 20, 1000
NC, NS, L = 2, 16, 16
NW = NC * NS
IPW = N0 // NW
RPW = IPW * N1
NBUF = 2


def _sc_body(idx_hbm, values_hbm, out_hbm, idx_v, vals_v, buf0, buf1, sems):
    c = lax.axis_index("c")
    s = lax.axis_index("s")
    w = s * NC + c
    base_i = w * IPW

    pltpu.sync_copy(idx_hbm.at[pl.ds(base_i * 32, IPW * 32)], idx_v)
    pltpu.sync_copy(values_hbm, vals_v)

    lanes = lax.iota(jnp.int32, 16)
    zeros16 = jnp.zeros((16,), jnp.int32)
    off_vec = vals_v[pl.ds(0, 16)]
    on_vec = vals_v[pl.ds(16, 16)]

    bufs = (buf0, buf1)

    for b in range(NBUF):
        for j in range(N1):
            def _fill(ci, _, b=b, j=j):
                bufs[b][0, j, pl.ds(ci * 16, 16)] = off_vec
                return _
            lax.fori_loop(0, K // 16, _fill, None)
            plsc.store_scatter(
                bufs[b],
                [zeros16, jnp.full((16,), j, jnp.int32), (K - 16) + lanes],
                off_vec,
            )

    j_a = jnp.minimum(lanes, N1 - 1)
    j_b = jnp.minimum(lanes + 16, N1 - 1)
    mask_a = lanes < 16
    mask_b = lanes < (N1 - 16)

    def _load_kv(si):
        lr0 = si * 32
        kv_a = idx_v[pl.ds(lr0, 16)]
        kv_b = idx_v[pl.ds(lr0 + 16, 16)]
        return jnp.minimum(kv_a, K - 1), jnp.minimum(kv_b, K - 1)

    def _poke(buf, si, val_vec):
        kv_a, kv_b = _load_kv(si)
        plsc.store_scatter(buf, [zeros16, j_a, kv_a], val_vec, mask=mask_a)
        plsc.store_scatter(buf, [zeros16, j_b, kv_b], val_vec, mask=mask_b)

    def _copy(buf, si, b):
        return pltpu.make_async_copy(
            buf, out_hbm.at[pl.ds(base_i + si, 1)], sems.at[b]
        )

    for b in range(NBUF):
        _poke(bufs[b], b, on_vec)
        _copy(bufs[b], b, b).start()

    def _step(g, _):
        for b in range(NBUF):
            si = g * NBUF + b
            buf = bufs[b]
            _copy(buf, si - NBUF, b).wait()
            _poke(buf, si - NBUF, off_vec)
            _poke(buf, si, on_vec)
            _copy(buf, si, b).start()
        return _

    lax.fori_loop(1, IPW // NBUF, _step, None)

    for b in range(NBUF):
        _copy(bufs[b], IPW - NBUF + b, b).wait()


_SC_PARAMS = pltpu.CompilerParams()
if "needs_layout_passes" in pltpu.CompilerParams.__dataclass_fields__:
    _SC_PARAMS = dataclasses.replace(_SC_PARAMS, needs_layout_passes=False)


def kernel(indices, values):
    fn = pl.kernel(
        _sc_body,
        out_type=jax.ShapeDtypeStruct((N0, N1, K), jnp.float32),
        mesh=plsc.VectorSubcoreMesh(core_axis_name="c", subcore_axis_name="s"),
        compiler_params=_SC_PARAMS,
        scratch_types=[
            pltpu.VMEM((IPW * 32,), jnp.int32),
            pltpu.VMEM((32,), jnp.float32),
            pltpu.VMEM((1, N1, K), jnp.float32),
            pltpu.VMEM((1, N1, K), jnp.float32),
            pltpu.SemaphoreType.DMA((NBUF,)),
        ],
    )
    values_splat = jnp.repeat(values, 16)
    idx_padded = jnp.pad(indices, ((0, 0), (0, 32 - N1))).reshape(N0 * 32)
    return fn(idx_padded, values_splat)


# speedup vs baseline: 1.0292x; 1.0087x over previous
import dataclasses

import jax
import jax.numpy as jnp
from jax import lax
from jax.experimental import pallas as pl
from jax.experimental.pallas import tpu as pltpu
from jax.experimental.pallas import tpu_sc as plsc

N0, N1, K = 4096, 20, 1000
NC, NS, L = 2, 16, 16
NW = NC * NS
IPW = N0 // NW
RPW = IPW * N1
NBUF = 2


def _sc_body(idx_hbm, values_hbm, out_hbm, idx_v, vals_v, buf0, buf1, sems):
    c = lax.axis_index("c")
    s = lax.axis_index("s")
    w = s * NC + c
    base_i = w * IPW

    pltpu.sync_copy(idx_hbm.at[pl.ds(base_i * 32, IPW * 32)], idx_v)
    pltpu.sync_copy(values_hbm, vals_v)

    lanes = lax.iota(jnp.int32, 16)
    zeros16 = jnp.zeros((16,), jnp.int32)
    off_vec = vals_v[pl.ds(0, 16)]
    on_vec = vals_v[pl.ds(16, 16)]

    bufs = (buf0, buf1)

    for b in range(NBUF):
        for j in range(N1):
            def _fill(ci, _, b=b, j=j):
                bufs[b][0, j, pl.ds(ci * 16, 16)] = off_vec
                return _
            lax.fori_loop(0, K // 16, _fill, None)
            plsc.store_scatter(
                bufs[b],
                [zeros16, jnp.full((16,), j, jnp.int32), (K - 16) + lanes],
                off_vec,
            )

    j_a = jnp.minimum(lanes, N1 - 1)
    j_b = jnp.minimum(lanes + 16, N1 - 1)
    mask_a = lanes < 16
    mask_b = lanes < (N1 - 16)

    def _load_kv(si):
        lr0 = si * 32
        kv_a = idx_v[pl.ds(lr0, 16)]
        kv_b = idx_v[pl.ds(lr0 + 16, 16)]
        return jnp.minimum(kv_a, K - 1), jnp.minimum(kv_b, K - 1)

    def _poke(buf, si, val_vec):
        kv_a, kv_b = _load_kv(si)
        plsc.store_scatter(buf, [zeros16, j_a, kv_a], val_vec, mask=mask_a)
        plsc.store_scatter(buf, [zeros16, j_b, kv_b], val_vec, mask=mask_b)

    def _copy(buf, si, b):
        return pltpu.make_async_copy(
            buf, out_hbm.at[pl.ds(base_i + si, 1)], sems.at[b]
        )

    for b in range(NBUF):
        _poke(bufs[b], b, on_vec)
        _copy(bufs[b], b, b).start()

    def _step(g, _):
        for b in range(NBUF):
            si = g * NBUF + b
            buf = bufs[b]
            _copy(buf, si - NBUF, b).wait()
            _poke(buf, si - NBUF, off_vec)
            _poke(buf, si, on_vec)
            _copy(buf, si, b).start()
        return _

    lax.fori_loop(1, IPW // NBUF, _step, None)

    for b in range(NBUF):
        _copy(bufs[b], IPW - NBUF + b, b).wait()


_SC_PARAMS = pltpu.CompilerParams()
if "needs_layout_passes" in pltpu.CompilerParams.__dataclass_fields__:
    _SC_PARAMS = dataclasses.replace(_SC_PARAMS, needs_layout_passes=False)
if "use_tc_tiling_on_sc" in pltpu.CompilerParams.__dataclass_fields__:
    _SC_PARAMS = dataclasses.replace(_SC_PARAMS, use_tc_tiling_on_sc=True)


def kernel(indices, values):
    fn = pl.kernel(
        _sc_body,
        out_type=jax.ShapeDtypeStruct((N0, N1, K), jnp.float32),
        mesh=plsc.VectorSubcoreMesh(core_axis_name="c", subcore_axis_name="s"),
        compiler_params=_SC_PARAMS,
        scratch_types=[
            pltpu.VMEM((IPW * 32,), jnp.int32),
            pltpu.VMEM((32,), jnp.float32),
            pltpu.VMEM((1, N1, K), jnp.float32),
            pltpu.VMEM((1, N1, K), jnp.float32),
            pltpu.SemaphoreType.DMA((NBUF,)),
        ],
    )
    values_splat = jnp.repeat(values, 16)
    idx_padded = jnp.pad(indices, ((0, 0), (0, 32 - N1))).reshape(N0 * 32)
    return fn(idx_padded, values_splat)


# speedup vs baseline: 4.6873x; 4.5544x over previous
import jax
import jax.numpy as jnp
from jax import lax
from jax.experimental import pallas as pl
from jax.experimental.pallas import tpu as pltpu

N0, N1, K = 4096, 20, 1000
KB = 200


def _onehot_t_body(values_ref, idx_ref, out_ref):
    off = values_ref[0]
    on = values_ref[1]
    idx = idx_ref[...]
    kk = lax.broadcasted_iota(jnp.int32, (1, KB, N0), 1) + pl.program_id(1) * KB
    out_ref[...] = jnp.where(kk == idx, on, off)


def kernel(indices, values):
    out_t = pl.pallas_call(
        _onehot_t_body,
        grid=(N1, K // KB),
        in_specs=[
            pl.BlockSpec(memory_space=pltpu.SMEM),
            pl.BlockSpec((1, 1, N0), lambda j, kb: (j, 0, 0)),
        ],
        out_specs=pl.BlockSpec((1, KB, N0), lambda j, kb: (j, kb, 0)),
        out_shape=jax.ShapeDtypeStruct((N1, K, N0), jnp.float32),
    )(values, indices.T.reshape(N1, 1, N0))
    return out_t.transpose(2, 0, 1)


# speedup vs baseline: 4.7601x; 1.0155x over previous
import jax
import jax.numpy as jnp
from jax import lax
from jax.experimental import pallas as pl
from jax.experimental.pallas import tpu as pltpu

N0, N1, K = 4096, 20, 1000
KB = 1000


def _onehot_t_body(values_ref, idx_ref, out_ref):
    off = values_ref[0]
    on = values_ref[1]
    idx = idx_ref[...]
    kk = lax.broadcasted_iota(jnp.int32, (1, KB, N0), 1) + pl.program_id(1) * KB
    out_ref[...] = jnp.where(kk == idx, on, off)


def kernel(indices, values):
    out_t = pl.pallas_call(
        _onehot_t_body,
        grid=(N1, K // KB),
        in_specs=[
            pl.BlockSpec(memory_space=pltpu.SMEM),
            pl.BlockSpec((1, 1, N0), lambda j, kb: (j, 0, 0)),
        ],
        out_specs=pl.BlockSpec((1, KB, N0), lambda j, kb: (j, kb, 0)),
        out_shape=jax.ShapeDtypeStruct((N1, K, N0), jnp.float32),
    )(values, indices.T.reshape(N1, 1, N0))
    return out_t.transpose(2, 0, 1)
